# baseline (speedup 1.0000x reference)
import functools

import jax
import jax.numpy as jnp
from jax import lax
from jax.experimental import pallas as pl
from jax.experimental.pallas import tpu as pltpu
from jax.experimental.pallas import tpu_sc as plsc

ATTRACTION_W = 1.0
REPULSION_W = 1.0
MARGIN = 0.3

B = 4
N = 65536
D = 64
NUM_IDS = 256
NLANE = 16
NWORK = 32
TPB = NWORK // B
CHUNK_A = N // TPB
CAP_T = 256
K = TPB * CAP_T

def _mesh():
    return plsc.VectorSubcoreMesh(core_axis_name="c", subcore_axis_name="s",
                                  num_cores=2, num_subcores=16)
def _dot(a, b, dims, prec=None):
    return lax.dot_general(a, b, (dims, ((), ())), precision=prec,
                           preferred_element_type=jnp.float32)


IPW = NUM_IDS // TPB
GCH = 16
GW = CAP_T // GCH


def _stage_ab_body(seg_hbm, cp_hbm, emb_hbm, beta_hbm,
                   fcp_red_out, tab_out, bcp_out, grows_out, tcnt_out,
                   seg_v, cp_v, fcp_tab, list_v, red_fcp, tcnt_v,
                   part_v, fcp_v, sidx_v, rows_v, brow_v, lidx2_v, grows_v,
                   shared_fcp, sem0, sem1, sem2):
    c = lax.axis_index("c")
    s = lax.axis_index("s")
    b = c * 2 + s // TPB
    t = s % TPB
    lo = t * CHUNK_A

    pltpu.sync_copy(seg_hbm.at[b, pl.ds(lo, CHUNK_A)], seg_v)
    pltpu.sync_copy(cp_hbm.at[b, pl.ds(lo, CHUNK_A)], cp_v)

    def _init(j, _):
        fcp_tab[pl.ds(j * NLANE, NLANE)] = jnp.full((NLANE,), N, jnp.int32)
        return 0
    lax.fori_loop(0, (NLANE * NUM_IDS) // NLANE, _init, 0)

    def _zlist(j, _):
        list_v[pl.ds(j * NLANE, NLANE)] = jnp.zeros((NLANE,), jnp.int32)
        return 0
    lax.fori_loop(0, CAP_T // NLANE, _zlist, 0)

    lane = lax.iota(jnp.int32, NLANE)

    def _scan(i, cc):
        base = i * NLANE
        c16 = cp_v[pl.ds(base, NLANE)] == 1
        hit = lo + base + lane
        plsc.store_compressed(list_v.at[pl.ds(cc, NLANE)], hit, mask=c16)
        return cc + plsc.all_reduce_population_count(c16)[0]

    cc = lax.fori_loop(0, CHUNK_A // NLANE, _scan, jnp.int32(0))

    def _upd(u, _):
        offs = u * NLANE + lane
        valid = offs < cc
        idx16 = list_v[pl.ds(u * NLANE, NLANE)]
        local = jnp.where(valid, idx16 - lo, 0)
        s16 = plsc.load_gather(seg_v, [local])
        tidx = lane * NUM_IDS + s16
        cur = plsc.load_gather(fcp_tab, [tidx])
        cand = jnp.where(valid, idx16, N)
        plsc.store_scatter(fcp_tab, [tidx], jnp.minimum(cur, cand))
        return 0
    lax.fori_loop(0, (cc + NLANE - 1) // NLANE, _upd, 0)

    def _red(j, _):
        o = j * NLANE
        m = fcp_tab[pl.ds(o, NLANE)]
        for l in range(1, NLANE):
            m = jnp.minimum(m, fcp_tab[pl.ds(l * NUM_IDS + o, NLANE)])
        red_fcp[pl.ds(o, NLANE)] = m
        return 0
    lax.fori_loop(0, NUM_IDS // NLANE, _red, 0)

    tcnt_v[...] = jnp.full((NLANE,), cc, jnp.int32)
    pltpu.sync_copy(tcnt_v, tcnt_out.at[b, t])

    pltpu.sync_copy(red_fcp, shared_fcp.at[s])
    plsc.subcore_barrier()

    o = t * IPW
    base_row = (s // TPB) * TPB
    with jax.named_scope("b_fcp_load"):
        for l in range(TPB):
            pltpu.sync_copy(shared_fcp.at[base_row + l, pl.ds(o, IPW)],
                            part_v.at[pl.ds(l * IPW, IPW)])
        for j in range(IPW // NLANE):
            m = part_v[pl.ds(j * NLANE, NLANE)]
            for l in range(1, TPB):
                m = jnp.minimum(m, part_v[pl.ds(l * IPW + j * NLANE, NLANE)])
            fcp_v[pl.ds(j * NLANE, NLANE)] = m
            sidx_v[pl.ds(j * NLANE, NLANE)] = jnp.minimum(m, N - 1)
        pltpu.sync_copy(fcp_v, fcp_red_out.at[b, pl.ds(o, IPW)])

    with jax.named_scope("b_list"):
        for j in range(GCH):
            for k in range(GW // NLANE):
                v = list_v[pl.ds(j * GW + k * NLANE, NLANE)]
                lidx2_v[j, pl.ds(k * NLANE, NLANE)] = jnp.clip(v, 0, N - 1)

    with jax.named_scope("b_gather"):
        cp1 = pltpu.async_copy(emb_hbm.at[b].at[sidx_v], rows_v, sem0)
        cp2 = pltpu.async_copy(beta_hbm.at[b].at[sidx_v], brow_v, sem1)
        gcps = [pltpu.async_copy(emb_hbm.at[b].at[lidx2_v.at[j]],
                                 grows_v.at[pl.ds(j * GW, GW)], sem2)
                for j in range(GCH)]
        cp1.wait()
        cp2.wait()
        for g in gcps:
            g.wait()
    with jax.named_scope("b_out"):
        pltpu.sync_copy(rows_v, tab_out.at[b, pl.ds(o, IPW)])
        pltpu.sync_copy(brow_v, bcp_out.at[b, pl.ds(o, IPW)])
        pltpu.sync_copy(grows_v, grows_out.at[b, pl.ds(t * CAP_T, CAP_T)])


def _stage_ab(seg, cp, emb, beta2):
    f32, i32 = jnp.float32, jnp.int32
    return pl.kernel(
        _stage_ab_body,
        out_type=(
            jax.ShapeDtypeStruct((B, NUM_IDS), i32),
            jax.ShapeDtypeStruct((B, NUM_IDS, D), f32),
            jax.ShapeDtypeStruct((B, NUM_IDS), f32),
            jax.ShapeDtypeStruct((B, K, D), f32),
            jax.ShapeDtypeStruct((B, TPB, NLANE), i32),
        ),
        mesh=_mesh(),
        compiler_params=pltpu.CompilerParams(needs_layout_passes=False, use_tc_tiling_on_sc=False),
        scratch_types=[
            pltpu.VMEM((CHUNK_A,), i32),
            pltpu.VMEM((CHUNK_A,), i32),
            pltpu.VMEM((NLANE * NUM_IDS,), i32),
            pltpu.VMEM((CHUNK_A,), i32),
            pltpu.VMEM((NUM_IDS,), i32),
            pltpu.VMEM((NLANE,), i32),
            pltpu.VMEM((TPB * IPW,), i32),
            pltpu.VMEM((IPW,), i32),
            pltpu.VMEM((IPW,), i32),
            pltpu.VMEM((IPW, D), f32),
            pltpu.VMEM((IPW,), f32),
            pltpu.VMEM((GCH, GW), i32),
            pltpu.VMEM((CAP_T, D), f32),
            pltpu.VMEM_SHARED((NLANE, NUM_IDS), i32),
            pltpu.SemaphoreType.DMA,
            pltpu.SemaphoreType.DMA,
            pltpu.SemaphoreType.DMA,
        ],
    )(seg, cp, emb, beta2)


CH = 2048
NCH = N // CH


SS = D + 4


def _stage_c_kernel(emb_ref, beta_ref, seg_ref, cp_ref, bcp_ref,
                    segsums_ref, scal_ref):
    c = pl.program_id(0)
    f32 = jnp.float32
    emb_all = emb_ref[...]
    beta_all = beta_ref[...]
    seg_all = seg_ref[...]
    cp_all = cp_ref[...]
    bcp_all = bcp_ref[...]

    ids_col = lax.broadcasted_iota(jnp.int32, (NUM_IDS, 1), 0)
    ones_d = jnp.ones((1, D), f32)

    @pl.when(c == 0)
    def _():
        segsums_ref[...] = jnp.zeros_like(segsums_ref)
        scal_ref[...] = jnp.zeros_like(scal_ref)

    rows = []
    for b in range(B):
        segb = seg_all[b:b + 1, :]
        oh = (ids_col == segb).astype(f32)
        embb = emb_all[b]
        e2 = embb * embb
        n_row = _dot(ones_d, e2, (((1,), (1,))))
        cpb = (cp_all[b:b + 1, :] == 1).astype(f32)
        betab = beta_all[b:b + 1, :]
        bcp_h = _dot(bcp_all[b:b + 1, :], oh, (((1,), (0,))))
        rterm = (jnp.maximum(betab + MARGIN - bcp_h, 0.0)
                 * (1.0 - cpb))
        rvals = jnp.concatenate(
            [n_row, jnp.ones_like(n_row), cpb, rterm], axis=0)
        ss_emb = _dot(oh, embb, (((1,), (0,))))
        ss_row = _dot(oh, rvals, (((1,), (1,))))
        segsums_ref[b, :, :] += jnp.concatenate([ss_emb, ss_row], axis=1)

        per = (jnp.maximum(betab, 0.0) - betab * cpb
               + jnp.log1p(jnp.exp(-jnp.abs(betab))))
        s_ncp = jnp.sum(per * (1.0 - cpb)).reshape(1, 1)
        s_cp = jnp.sum(per * cpb).reshape(1, 1)
        rows.append(jnp.concatenate([s_ncp, s_cp], axis=1))

    scal_ref[...] += jnp.concatenate(rows, axis=0)


def _stage_c(emb, beta2, seg, cp, bcp):
    f32 = jnp.float32
    return pl.pallas_call(
        _stage_c_kernel,
        grid=(NCH,),
        in_specs=[
            pl.BlockSpec((B, CH, D), lambda c: (0, c, 0)),
            pl.BlockSpec((B, CH), lambda c: (0, c)),
            pl.BlockSpec((B, CH), lambda c: (0, c)),
            pl.BlockSpec((B, CH), lambda c: (0, c)),
            pl.BlockSpec((B, NUM_IDS), lambda c: (0, 0)),
        ],
        out_specs=[
            pl.BlockSpec((B, NUM_IDS, SS), lambda c: (0, 0, 0)),
            pl.BlockSpec((B, 2), lambda c: (0, 0)),
        ],
        out_shape=[
            jax.ShapeDtypeStruct((B, NUM_IDS, SS), f32),
            jax.ShapeDtypeStruct((B, 2), f32),
        ],
    )(emb, beta2, seg, cp, bcp)


RB = 512


def _pair_block(rows, cols, wrow, wcol, acc_ref, is_first):
    rn = jnp.sum(rows * rows, axis=1, keepdims=True)
    cn = jnp.sum(cols * cols, axis=1, keepdims=True)
    cn = cn + (1.0 - wcol) * jnp.float32(1e9)
    ones = jnp.ones((1, 1), jnp.float32)
    cnrow = _dot(ones, cn, (((1,), (1,))), prec=jax.lax.Precision.HIGHEST)
    dots = lax.dot_general(rows, cols, ((((1,), (1,))), ((), ())),
                           preferred_element_type=jnp.float32)
    d2p = jnp.maximum(rn + cnrow - 2.0 * dots, 0.0)
    s = jnp.sum(jnp.exp(-d2p) * wrow)
    bc = jnp.broadcast_to(s.reshape(1, 1, 1), (1, 1, 128))

    @pl.when(is_first)
    def _():
        acc_ref[...] = jnp.zeros_like(acc_ref)

    acc_ref[...] += bc


def _stage_d_kernel(g_rows_ref, g_cols_ref, tcnt_ref, pair_ref):
    r = pl.program_id(1)
    rows = g_rows_ref[0]
    cols = g_cols_ref[0]
    tc = tcnt_ref[0][:, 0:1].astype(jnp.float32)

    def w_of(k0, n):
        kk = k0 + lax.broadcasted_iota(jnp.int32, (n, 1), 0)
        oh = (kk // CAP_T == lax.broadcasted_iota(jnp.int32, (1, TPB), 1))
        lim = _dot(oh.astype(jnp.float32), tc, (((1,), (0,))))
        return ((kk % CAP_T).astype(jnp.float32) < lim).astype(jnp.float32)

    wrow = w_of(r * RB, RB)
    wcol = w_of(0, K)
    _pair_block(rows, cols, wrow, wcol, pair_ref, r == 0)


def _stage_d(g, tcnt):
    return pl.pallas_call(
        _stage_d_kernel,
        grid=(B, K // RB),
        in_specs=[
            pl.BlockSpec((1, RB, D), lambda b, r: (b, r, 0)),
            pl.BlockSpec((1, K, D), lambda b, r: (b, 0, 0)),
            pl.BlockSpec((1, TPB, NLANE), lambda b, r: (b, 0, 0)),
        ],
        out_specs=pl.BlockSpec((1, 1, 128), lambda b, r: (b, 0, 0)),
        out_shape=jax.ShapeDtypeStruct((B, 1, 128), jnp.float32),
    )(g, g, tcnt)


DCH = 2048


def _stage_d_dense_kernel(rows_ref, cols_ref, cpr_ref, cpc_ref, pair_ref):
    r = pl.program_id(1)
    c = pl.program_id(2)
    wrow = (cpr_ref[0] == 1).astype(jnp.float32)
    wcol = (cpc_ref[0] == 1).astype(jnp.float32)
    _pair_block(rows_ref[0], cols_ref[0], wrow, wcol, pair_ref,
                (r == 0) & (c == 0))


def _stage_d_dense(emb, cp3):
    return pl.pallas_call(
        _stage_d_dense_kernel,
        grid=(B, N // DCH, N // DCH),
        in_specs=[
            pl.BlockSpec((1, DCH, D), lambda b, r, c: (b, r, 0)),
            pl.BlockSpec((1, DCH, D), lambda b, r, c: (b, c, 0)),
            pl.BlockSpec((1, DCH, 1), lambda b, r, c: (b, r, 0)),
            pl.BlockSpec((1, DCH, 1), lambda b, r, c: (b, c, 0)),
        ],
        out_specs=pl.BlockSpec((1, 1, 128), lambda b, r, c: (b, 0, 0)),
        out_shape=jax.ShapeDtypeStruct((B, 1, 128), jnp.float32),
    )(emb, emb, cp3, cp3)


def _stage_e_kernel(fcp_ref, tab_ref, segsums_ref, scal_ref, pair_ref,
                    out_ref):
    f32 = jnp.float32
    fcp = fcp_ref[...][:, 0, :]
    segsums = segsums_ref[...]
    esum = segsums[:, :, 0:D]
    nsum = segsums[:, :, D]
    cntf = segsums[:, :, D + 1]
    cpcf = segsums[:, :, D + 2]
    sum_rank = segsums[:, :, D + 3]
    ncpf = cntf - cpcf

    tab = tab_ref[...]
    q = jnp.sum(tab * tab, axis=2)
    cross = jnp.sum(tab * esum, axis=2)
    sum_d2 = nsum - 2.0 * cross + cntf * q

    has = (fcp < N).astype(f32)
    pos = jnp.sum(cpcf, axis=1, keepdims=True)
    neg = jnp.float32(N) - pos
    pw = neg / (pos + 1e-6)

    scal = scal_ref[...]
    s_ncp = scal[:, 0:1]
    s_cp = scal[:, 1:2]
    beta_ce = (s_ncp + pw * s_cp) * jnp.float32(1.0 / N)

    mean_d2 = sum_d2 / jnp.maximum(cntf, 1.0)
    attraction = jnp.sum(has * mean_d2, axis=1, keepdims=True) * ATTRACTION_W

    mean_rank = sum_rank / jnp.maximum(ncpf, 1.0)
    rmask = ((cpcf == 1.0) & (ncpf > 0.0)).astype(f32)
    num_unique = jnp.sum((cntf > 0.0).astype(f32), axis=1, keepdims=True)
    ranking = (jnp.sum(rmask * mean_rank, axis=1, keepdims=True)
               / jnp.maximum(num_unique, 1.0))
    beta_loss = beta_ce + 2.0 * ranking

    pair = pair_ref[...][:, 0, 0:1]
    repulsion = jnp.where(pos > 1.0,
                          pair / jnp.maximum(pos * pos, 1.0),
                          0.0) * REPULSION_W

    loss = beta_loss + attraction + repulsion
    inc = ((pos >= 1.0) & (neg >= 1.0)).astype(f32)
    count = jnp.sum(inc)
    den = jnp.maximum(count, 1.0)
    total = jnp.sum(inc * loss) / den
    final = jnp.where(count > 0.0, total, 0.0)
    blog = jnp.sum(inc * beta_loss) / den
    alog = jnp.sum(inc * attraction) / den
    rlog = jnp.sum(inc * repulsion) / den

    vec = jnp.concatenate(
        [final.reshape(1, 1), blog.reshape(1, 1), alog.reshape(1, 1),
         rlog.reshape(1, 1), jnp.zeros((1, 4), f32)], axis=1)
    out_ref[...] = vec.reshape(1, 1, 8)


def _stage_e(fcp_red3, tab, segsums, scal, pair):
    return pl.pallas_call(
        _stage_e_kernel,
        out_shape=jax.ShapeDtypeStruct((1, 1, 8), jnp.float32),
    )(fcp_red3, tab, segsums, scal, pair)


@jax.jit
def kernel(beta, embed, slice_id, is_cp):
    f32, i32 = jnp.float32, jnp.int32
    emb = embed.astype(f32)
    beta2 = jnp.squeeze(beta, -1).astype(f32)
    seg = slice_id.astype(i32)
    cp = is_cp.astype(i32)

    fcp_red, tab, bcp, g, tcnt = _stage_ab(seg, cp, emb, beta2)

    segsums, scal = _stage_c(emb, beta2, seg, cp, bcp)

    overflow = jnp.any(tcnt[:, :, 0] > CAP_T)
    pair = lax.cond(
        overflow,
        lambda e, c2, gg, tc: _stage_d_dense(e, c2[..., None]),
        lambda e, c2, gg, tc: _stage_d(gg, tc),
        emb, cp, g, tcnt)

    out = _stage_e(fcp_red[:, None, :], tab, segsums, scal, pair)
    final_loss = out[0, 0, 0]
    extras = {'beta_loss': out[0, 0, 1],
              'attr_loss': out[0, 0, 2],
              'repl_loss': out[0, 0, 3]}
    return final_loss, extras


# speedup vs baseline: 1.0039x; 1.0039x over previous
import jax
import jax.numpy as jnp
from jax import lax
from jax.experimental import pallas as pl
from jax.experimental.pallas import tpu as pltpu
from jax.experimental.pallas import tpu_sc as plsc

ATTRACTION_W = 1.0
REPULSION_W = 1.0
MARGIN = 0.3

B = 4
N = 65536
D = 64
NUM_IDS = 256
NLANE = 16
NWORK = 32
TPB = NWORK // B
CHUNK_A = N // TPB
CAP_T = 256
K = TPB * CAP_T

def _mesh():
    return plsc.VectorSubcoreMesh(core_axis_name="c", subcore_axis_name="s",
                                  num_cores=2, num_subcores=16)
def _dot(a, b, dims, prec=None):
    return lax.dot_general(a, b, (dims, ((), ())), precision=prec,
                           preferred_element_type=jnp.float32)


IPW = NUM_IDS // TPB
GCH = 8
GW = CAP_T // GCH


def _stage_ab_body(seg_hbm, cp_hbm, emb_hbm, beta_hbm,
                   fcp_red_out, tab_out, bcp_out, grows_out, tcnt_out,
                   seg_v, cp_v, fcp_tab, list_v, red_fcp, tcnt_v,
                   part_v, fcp_v, sidx_v, rows_v, brow_v, lidx2_v, grows_v,
                   shared_fcp, sem0, sem1, sem2):
    c = lax.axis_index("c")
    s = lax.axis_index("s")
    b = c * 2 + s // TPB
    t = s % TPB
    lo = t * CHUNK_A

    pltpu.sync_copy(seg_hbm.at[b, pl.ds(lo, CHUNK_A)], seg_v)
    pltpu.sync_copy(cp_hbm.at[b, pl.ds(lo, CHUNK_A)], cp_v)

    def _init(j, _):
        fcp_tab[pl.ds(j * NLANE, NLANE)] = jnp.full((NLANE,), N, jnp.int32)
        return 0
    lax.fori_loop(0, (NLANE * NUM_IDS) // NLANE, _init, 0)

    def _zlist(j, _):
        list_v[pl.ds(j * NLANE, NLANE)] = jnp.zeros((NLANE,), jnp.int32)
        return 0
    lax.fori_loop(0, CAP_T // NLANE, _zlist, 0)

    lane = lax.iota(jnp.int32, NLANE)

    def _scan(i, cc):
        base = i * NLANE
        c16 = cp_v[pl.ds(base, NLANE)] == 1
        hit = lo + base + lane
        plsc.store_compressed(list_v.at[pl.ds(cc, NLANE)], hit, mask=c16)
        return cc + plsc.all_reduce_population_count(c16)[0]

    cc = lax.fori_loop(0, CHUNK_A // NLANE, _scan, jnp.int32(0))

    def _upd(u, _):
        offs = u * NLANE + lane
        valid = offs < cc
        idx16 = list_v[pl.ds(u * NLANE, NLANE)]
        local = jnp.where(valid, idx16 - lo, 0)
        s16 = plsc.load_gather(seg_v, [local])
        tidx = lane * NUM_IDS + s16
        cur = plsc.load_gather(fcp_tab, [tidx])
        cand = jnp.where(valid, idx16, N)
        plsc.store_scatter(fcp_tab, [tidx], jnp.minimum(cur, cand))
        return 0
    lax.fori_loop(0, (cc + NLANE - 1) // NLANE, _upd, 0)

    def _red(j, _):
        o = j * NLANE
        m = fcp_tab[pl.ds(o, NLANE)]
        for l in range(1, NLANE):
            m = jnp.minimum(m, fcp_tab[pl.ds(l * NUM_IDS + o, NLANE)])
        red_fcp[pl.ds(o, NLANE)] = m
        return 0
    lax.fori_loop(0, NUM_IDS // NLANE, _red, 0)

    tcnt_v[...] = jnp.full((NLANE,), cc, jnp.int32)
    pltpu.sync_copy(tcnt_v, tcnt_out.at[b, t])

    pltpu.sync_copy(red_fcp, shared_fcp.at[s])
    plsc.subcore_barrier()

    o = t * IPW
    base_row = (s // TPB) * TPB
    with jax.named_scope("b_fcp_load"):
        for l in range(TPB):
            pltpu.sync_copy(shared_fcp.at[base_row + l, pl.ds(o, IPW)],
                            part_v.at[pl.ds(l * IPW, IPW)])
        for j in range(IPW // NLANE):
            m = part_v[pl.ds(j * NLANE, NLANE)]
            for l in range(1, TPB):
                m = jnp.minimum(m, part_v[pl.ds(l * IPW + j * NLANE, NLANE)])
            fcp_v[pl.ds(j * NLANE, NLANE)] = m
            sidx_v[pl.ds(j * NLANE, NLANE)] = jnp.minimum(m, N - 1)
        pltpu.sync_copy(fcp_v, fcp_red_out.at[b, pl.ds(o, IPW)])

    with jax.named_scope("b_list"):
        for j in range(GCH):
            for k in range(GW // NLANE):
                v = list_v[pl.ds(j * GW + k * NLANE, NLANE)]
                lidx2_v[j, pl.ds(k * NLANE, NLANE)] = jnp.clip(v, 0, N - 1)

    with jax.named_scope("b_gather"):
        cp1 = pltpu.async_copy(emb_hbm.at[b].at[sidx_v], rows_v, sem0)
        cp2 = pltpu.async_copy(beta_hbm.at[b].at[sidx_v], brow_v, sem1)
        gcps = [pltpu.async_copy(emb_hbm.at[b].at[lidx2_v.at[j]],
                                 grows_v.at[pl.ds(j * GW, GW)], sem2)
                for j in range(GCH)]
        cp1.wait()
        cp2.wait()
        for g in gcps:
            g.wait()
    with jax.named_scope("b_out"):
        pltpu.sync_copy(rows_v, tab_out.at[b, pl.ds(o, IPW)])
        pltpu.sync_copy(brow_v, bcp_out.at[b, pl.ds(o, IPW)])
        pltpu.sync_copy(grows_v, grows_out.at[b, pl.ds(t * CAP_T, CAP_T)])


def _stage_ab(seg, cp, emb, beta2):
    f32, i32 = jnp.float32, jnp.int32
    return pl.kernel(
        _stage_ab_body,
        out_type=(
            jax.ShapeDtypeStruct((B, NUM_IDS), i32),
            jax.ShapeDtypeStruct((B, NUM_IDS, D), f32),
            jax.ShapeDtypeStruct((B, NUM_IDS), f32),
            jax.ShapeDtypeStruct((B, K, D), f32),
            jax.ShapeDtypeStruct((B, TPB, NLANE), i32),
        ),
        mesh=_mesh(),
        compiler_params=pltpu.CompilerParams(needs_layout_passes=False, use_tc_tiling_on_sc=False),
        scratch_types=[
            pltpu.VMEM((CHUNK_A,), i32),
            pltpu.VMEM((CHUNK_A,), i32),
            pltpu.VMEM((NLANE * NUM_IDS,), i32),
            pltpu.VMEM((CHUNK_A,), i32),
            pltpu.VMEM((NUM_IDS,), i32),
            pltpu.VMEM((NLANE,), i32),
            pltpu.VMEM((TPB * IPW,), i32),
            pltpu.VMEM((IPW,), i32),
            pltpu.VMEM((IPW,), i32),
            pltpu.VMEM((IPW, D), f32),
            pltpu.VMEM((IPW,), f32),
            pltpu.VMEM((GCH, GW), i32),
            pltpu.VMEM((CAP_T, D), f32),
            pltpu.VMEM_SHARED((NLANE, NUM_IDS), i32),
            pltpu.SemaphoreType.DMA,
            pltpu.SemaphoreType.DMA,
            pltpu.SemaphoreType.DMA,
        ],
    )(seg, cp, emb, beta2)


CH = 2048
NCH = N // CH


SS = D + 4


def _stage_c_kernel(emb_ref, beta_ref, seg_ref, cp_ref, bcp_ref,
                    segsums_ref, scal_ref):
    c = pl.program_id(0)
    f32 = jnp.float32
    emb_all = emb_ref[...]
    beta_all = beta_ref[...]
    seg_all = seg_ref[...]
    cp_all = cp_ref[...]
    bcp_all = bcp_ref[...]

    ids_col = lax.broadcasted_iota(jnp.int32, (NUM_IDS, 1), 0)
    ones_d = jnp.ones((1, D), f32)

    @pl.when(c == 0)
    def _():
        segsums_ref[...] = jnp.zeros_like(segsums_ref)
        scal_ref[...] = jnp.zeros_like(scal_ref)

    rows = []
    for b in range(B):
        segb = seg_all[b:b + 1, :]
        oh = (ids_col == segb).astype(f32)
        embb = emb_all[b]
        e2 = embb * embb
        n_row = _dot(ones_d, e2, (((1,), (1,))))
        cpb = (cp_all[b:b + 1, :] == 1).astype(f32)
        betab = beta_all[b:b + 1, :]
        bcp_h = _dot(bcp_all[b:b + 1, :], oh, (((1,), (0,))))
        rterm = (jnp.maximum(betab + MARGIN - bcp_h, 0.0)
                 * (1.0 - cpb))
        rvals = jnp.concatenate(
            [n_row, jnp.ones_like(n_row), cpb, rterm], axis=0)
        ss_emb = _dot(oh, embb, (((1,), (0,))))
        ss_row = _dot(oh, rvals, (((1,), (1,))))
        segsums_ref[b, :, :] += jnp.concatenate([ss_emb, ss_row], axis=1)

        per = (jnp.maximum(betab, 0.0) - betab * cpb
               + jnp.log1p(jnp.exp(-jnp.abs(betab))))
        s_ncp = jnp.sum(per * (1.0 - cpb)).reshape(1, 1)
        s_cp = jnp.sum(per * cpb).reshape(1, 1)
        rows.append(jnp.concatenate([s_ncp, s_cp], axis=1))

    scal_ref[...] += jnp.concatenate(rows, axis=0)


def _stage_c(emb, beta2, seg, cp, bcp):
    f32 = jnp.float32
    return pl.pallas_call(
        _stage_c_kernel,
        grid=(NCH,),
        in_specs=[
            pl.BlockSpec((B, CH, D), lambda c: (0, c, 0)),
            pl.BlockSpec((B, CH), lambda c: (0, c)),
            pl.BlockSpec((B, CH), lambda c: (0, c)),
            pl.BlockSpec((B, CH), lambda c: (0, c)),
            pl.BlockSpec((B, NUM_IDS), lambda c: (0, 0)),
        ],
        out_specs=[
            pl.BlockSpec((B, NUM_IDS, SS), lambda c: (0, 0, 0)),
            pl.BlockSpec((B, 2), lambda c: (0, 0)),
        ],
        out_shape=[
            jax.ShapeDtypeStruct((B, NUM_IDS, SS), f32),
            jax.ShapeDtypeStruct((B, 2), f32),
        ],
    )(emb, beta2, seg, cp, bcp)


RB = 512


def _pair_block(rows, cols, wrow, wcol, acc_ref, is_first):
    rn = jnp.sum(rows * rows, axis=1, keepdims=True)
    cn = jnp.sum(cols * cols, axis=1, keepdims=True)
    cn = cn + (1.0 - wcol) * jnp.float32(1e9)
    ones = jnp.ones((1, 1), jnp.float32)
    cnrow = _dot(ones, cn, (((1,), (1,))), prec=jax.lax.Precision.HIGHEST)
    dots = lax.dot_general(rows, cols, ((((1,), (1,))), ((), ())),
                           preferred_element_type=jnp.float32)
    d2p = jnp.maximum(rn + cnrow - 2.0 * dots, 0.0)
    s = jnp.sum(jnp.exp(-d2p) * wrow)
    bc = jnp.broadcast_to(s.reshape(1, 1, 1), (1, 1, 128))

    @pl.when(is_first)
    def _():
        acc_ref[...] = jnp.zeros_like(acc_ref)

    acc_ref[...] += bc


def _stage_d_kernel(g_rows_ref, g_cols_ref, tcnt_ref, pair_ref):
    r = pl.program_id(1)
    rows = g_rows_ref[0]
    cols = g_cols_ref[0]
    tc = tcnt_ref[0][:, 0:1].astype(jnp.float32)

    def w_of(k0, n):
        kk = k0 + lax.broadcasted_iota(jnp.int32, (n, 1), 0)
        oh = (kk // CAP_T == lax.broadcasted_iota(jnp.int32, (1, TPB), 1))
        lim = _dot(oh.astype(jnp.float32), tc, (((1,), (0,))))
        return ((kk % CAP_T).astype(jnp.float32) < lim).astype(jnp.float32)

    wrow = w_of(r * RB, RB)
    wcol = w_of(0, K)
    _pair_block(rows, cols, wrow, wcol, pair_ref, r == 0)


def _stage_d(g, tcnt):
    return pl.pallas_call(
        _stage_d_kernel,
        grid=(B, K // RB),
        in_specs=[
            pl.BlockSpec((1, RB, D), lambda b, r: (b, r, 0)),
            pl.BlockSpec((1, K, D), lambda b, r: (b, 0, 0)),
            pl.BlockSpec((1, TPB, NLANE), lambda b, r: (b, 0, 0)),
        ],
        out_specs=pl.BlockSpec((1, 1, 128), lambda b, r: (b, 0, 0)),
        out_shape=jax.ShapeDtypeStruct((B, 1, 128), jnp.float32),
    )(g, g, tcnt)


DCH = 2048


def _stage_d_dense_kernel(rows_ref, cols_ref, cpr_ref, cpc_ref, pair_ref):
    r = pl.program_id(1)
    c = pl.program_id(2)
    wrow = (cpr_ref[0] == 1).astype(jnp.float32)
    wcol = (cpc_ref[0] == 1).astype(jnp.float32)
    _pair_block(rows_ref[0], cols_ref[0], wrow, wcol, pair_ref,
                (r == 0) & (c == 0))


def _stage_d_dense(emb, cp3):
    return pl.pallas_call(
        _stage_d_dense_kernel,
        grid=(B, N // DCH, N // DCH),
        in_specs=[
            pl.BlockSpec((1, DCH, D), lambda b, r, c: (b, r, 0)),
            pl.BlockSpec((1, DCH, D), lambda b, r, c: (b, c, 0)),
            pl.BlockSpec((1, DCH, 1), lambda b, r, c: (b, r, 0)),
            pl.BlockSpec((1, DCH, 1), lambda b, r, c: (b, c, 0)),
        ],
        out_specs=pl.BlockSpec((1, 1, 128), lambda b, r, c: (b, 0, 0)),
        out_shape=jax.ShapeDtypeStruct((B, 1, 128), jnp.float32),
    )(emb, emb, cp3, cp3)


def _stage_e_kernel(fcp_ref, tab_ref, segsums_ref, scal_ref, pair_ref,
                    out_ref):
    f32 = jnp.float32
    fcp = fcp_ref[...][:, 0, :]
    segsums = segsums_ref[...]
    esum = segsums[:, :, 0:D]
    nsum = segsums[:, :, D]
    cntf = segsums[:, :, D + 1]
    cpcf = segsums[:, :, D + 2]
    sum_rank = segsums[:, :, D + 3]
    ncpf = cntf - cpcf

    tab = tab_ref[...]
    q = jnp.sum(tab * tab, axis=2)
    cross = jnp.sum(tab * esum, axis=2)
    sum_d2 = nsum - 2.0 * cross + cntf * q

    has = (fcp < N).astype(f32)
    pos = jnp.sum(cpcf, axis=1, keepdims=True)
    neg = jnp.float32(N) - pos
    pw = neg / (pos + 1e-6)

    scal = scal_ref[...]
    s_ncp = scal[:, 0:1]
    s_cp = scal[:, 1:2]
    beta_ce = (s_ncp + pw * s_cp) * jnp.float32(1.0 / N)

    mean_d2 = sum_d2 / jnp.maximum(cntf, 1.0)
    attraction = jnp.sum(has * mean_d2, axis=1, keepdims=True) * ATTRACTION_W

    mean_rank = sum_rank / jnp.maximum(ncpf, 1.0)
    rmask = ((cpcf == 1.0) & (ncpf > 0.0)).astype(f32)
    num_unique = jnp.sum((cntf > 0.0).astype(f32), axis=1, keepdims=True)
    ranking = (jnp.sum(rmask * mean_rank, axis=1, keepdims=True)
               / jnp.maximum(num_unique, 1.0))
    beta_loss = beta_ce + 2.0 * ranking

    pair = pair_ref[...][:, 0, 0:1]
    repulsion = jnp.where(pos > 1.0,
                          pair / jnp.maximum(pos * pos, 1.0),
                          0.0) * REPULSION_W

    loss = beta_loss + attraction + repulsion
    inc = ((pos >= 1.0) & (neg >= 1.0)).astype(f32)
    count = jnp.sum(inc)
    den = jnp.maximum(count, 1.0)
    total = jnp.sum(inc * loss) / den
    final = jnp.where(count > 0.0, total, 0.0)
    blog = jnp.sum(inc * beta_loss) / den
    alog = jnp.sum(inc * attraction) / den
    rlog = jnp.sum(inc * repulsion) / den

    vec = jnp.concatenate(
        [final.reshape(1, 1), blog.reshape(1, 1), alog.reshape(1, 1),
         rlog.reshape(1, 1), jnp.zeros((1, 4), f32)], axis=1)
    out_ref[...] = vec.reshape(1, 1, 8)


def _stage_e(fcp_red3, tab, segsums, scal, pair):
    return pl.pallas_call(
        _stage_e_kernel,
        out_shape=jax.ShapeDtypeStruct((1, 1, 8), jnp.float32),
    )(fcp_red3, tab, segsums, scal, pair)


@jax.jit
def kernel(beta, embed, slice_id, is_cp):
    f32, i32 = jnp.float32, jnp.int32
    emb = embed.astype(f32)
    beta2 = jnp.squeeze(beta, -1).astype(f32)
    seg = slice_id.astype(i32)
    cp = is_cp.astype(i32)

    fcp_red, tab, bcp, g, tcnt = _stage_ab(seg, cp, emb, beta2)

    segsums, scal = _stage_c(emb, beta2, seg, cp, bcp)

    overflow = jnp.any(tcnt[:, :, 0] > CAP_T)
    pair = lax.cond(
        overflow,
        lambda e, c2, gg, tc: _stage_d_dense(e, c2[..., None]),
        lambda e, c2, gg, tc: _stage_d(gg, tc),
        emb, cp, g, tcnt)

    out = _stage_e(fcp_red[:, None, :], tab, segsums, scal, pair)
    final_loss = out[0, 0, 0]
    extras = {'beta_loss': out[0, 0, 1],
              'attr_loss': out[0, 0, 2],
              'repl_loss': out[0, 0, 3]}
    return final_loss, extras
